# baseline (speedup 1.0000x reference)
import jax
import jax.numpy as jnp
from jax import lax
from jax.experimental import pallas as pl
from jax.experimental.pallas import tpu as pltpu

DIM = 192
NPOS = 196
NROW = 14
WINL = 384
NB = 8
_LCHUNKS = ((0, 128), (128, 128), (256, 128))


def _body(s_ref, n_ref, nft_ref, pm_ref, out_ref, buf, sem):
    i = pl.program_id(0)
    ngroups = pl.num_programs(0)

    def issue(g, slot, wait):
        for n in range(NB):
            b = g * NB + n
            start = s_ref[b]
            astart = (start // 128) * 128
            need = start - astart + n_ref[b]
            for c0, clen in _LCHUNKS:
                cp = pltpu.make_async_copy(
                    nft_ref.at[:, pl.ds(astart + c0, clen)],
                    buf.at[slot, n, :, pl.ds(c0, clen)],
                    sem.at[slot, n],
                )

                @pl.when(
                    jnp.logical_or(g < 2, jnp.logical_or(c0 == 0, c0 < need))
                )
                def _():
                    if wait:
                        cp.wait()
                    else:
                        cp.start()

    @pl.when(i == 0)
    def _():
        issue(0, 0, wait=False)

    @pl.when(i + 1 < ngroups)
    def _():
        issue(i + 1, (i + 1) % 2, wait=False)

    issue(i, i % 2, wait=True)

    slot = i % 2
    s_lane = lax.broadcasted_iota(jnp.int32, (1, NPOS), 1)
    pm = pm_ref[...]
    for n in range(NB):
        b = i * NB + n
        start = s_ref[b]
        o = start - (start // 128) * 128
        w = buf[slot, n]
        g1 = pltpu.roll(w, lax.rem(WINL - o, WINL), axis=1)
        t = g1[:, :NPOS]
        t = t * (s_lane < n_ref[b]).astype(jnp.float32)
        o_b = lax.dot_general(
            t, pm, (((1,), (0,)), ((), ())), preferred_element_type=jnp.float32
        )
        out_ref[n] = o_b


def kernel(node_features, num_sp_list):
    ns = num_sp_list.astype(jnp.int32)
    starts = (jnp.cumsum(ns) - ns).astype(jnp.int32)
    B = ns.shape[0]
    nft = node_features.T
    jj = jnp.arange(NPOS, dtype=jnp.int32)
    pj = (jj % NROW) * NROW + jj // NROW
    pm = (jnp.arange(NPOS, dtype=jnp.int32)[:, None] == pj[None, :]).astype(
        jnp.float32
    )
    grid_spec = pltpu.PrefetchScalarGridSpec(
        num_scalar_prefetch=2,
        grid=(B // NB,),
        in_specs=[
            pl.BlockSpec(memory_space=pl.ANY),
            pl.BlockSpec((NPOS, NPOS), lambda i, s_ref, n_ref: (0, 0)),
        ],
        out_specs=pl.BlockSpec((NB, DIM, NPOS), lambda i, s_ref, n_ref: (i, 0, 0)),
        scratch_shapes=[
            pltpu.VMEM((2, NB, DIM, WINL), jnp.float32),
            pltpu.SemaphoreType.DMA((2, NB)),
        ],
    )
    out = pl.pallas_call(
        _body,
        grid_spec=grid_spec,
        out_shape=jax.ShapeDtypeStruct((B, DIM, NPOS), jnp.float32),
    )(starts, ns, nft, pm)
    return out.reshape(B, DIM, NROW, NROW)


# speedup vs baseline: 1.1162x; 1.1162x over previous
import jax
import jax.numpy as jnp
from jax import lax
from jax.experimental import pallas as pl
from jax.experimental.pallas import tpu as pltpu

DIM = 192
NPOS = 196
NROW = 14
NB = 8
GWIN = 1920
WINL = 384
_NCHUNK = GWIN // 128


def _body(s_ref, n_ref, nft_ref, pm_ref, out_ref, buf, sem):
    i = pl.program_id(0)
    ngroups = pl.num_programs(0)

    def issue(g, slot, wait):
        first = g * NB
        gbase = (s_ref[first] // 128) * 128
        need = (
            s_ref[first]
            - gbase
            + s_ref[first + NB - 1]
            - s_ref[first]
            + n_ref[first + NB - 1]
        )
        for c in range(_NCHUNK):
            cp = pltpu.make_async_copy(
                nft_ref.at[:, pl.ds(gbase + c * 128, 128)],
                buf.at[slot, :, pl.ds(c * 128, 128)],
                sem.at[slot],
            )

            @pl.when(jnp.logical_or(g < 2, jnp.logical_or(c == 0, c * 128 < need)))
            def _():
                if wait:
                    cp.wait()
                else:
                    cp.start()

    @pl.when(i == 0)
    def _():
        issue(0, 0, wait=False)

    @pl.when(i + 1 < ngroups)
    def _():
        issue(i + 1, (i + 1) % 2, wait=False)

    issue(i, i % 2, wait=True)

    slot = i % 2
    s_lane = lax.broadcasted_iota(jnp.int32, (1, NPOS), 1)
    pm = pm_ref[...]
    gbase128 = s_ref[i * NB] // 128
    for n in range(NB):
        b = i * NB + n
        start = s_ref[b]
        rel = (start // 128 - gbase128) * 128
        o = start - (start // 128) * 128
        w = buf[slot, :, pl.ds(rel, WINL)]
        g1 = pltpu.roll(w, lax.rem(WINL - o, WINL), axis=1)
        t = g1[:, :NPOS]
        t = t * (s_lane < n_ref[b]).astype(jnp.float32)
        o_b = lax.dot_general(
            t, pm, (((1,), (0,)), ((), ())), preferred_element_type=jnp.float32
        )
        out_ref[n] = o_b


def kernel(node_features, num_sp_list):
    ns = num_sp_list.astype(jnp.int32)
    starts = (jnp.cumsum(ns) - ns).astype(jnp.int32)
    B = ns.shape[0]
    nft = node_features.T
    jj = jnp.arange(NPOS, dtype=jnp.int32)
    pj = (jj % NROW) * NROW + jj // NROW
    pm = (jnp.arange(NPOS, dtype=jnp.int32)[:, None] == pj[None, :]).astype(
        jnp.float32
    )
    grid_spec = pltpu.PrefetchScalarGridSpec(
        num_scalar_prefetch=2,
        grid=(B // NB,),
        in_specs=[
            pl.BlockSpec(memory_space=pl.ANY),
            pl.BlockSpec((NPOS, NPOS), lambda i, s_ref, n_ref: (0, 0)),
        ],
        out_specs=pl.BlockSpec((NB, DIM, NPOS), lambda i, s_ref, n_ref: (i, 0, 0)),
        scratch_shapes=[
            pltpu.VMEM((2, DIM, GWIN), jnp.float32),
            pltpu.SemaphoreType.DMA((2,)),
        ],
    )
    out = pl.pallas_call(
        _body,
        grid_spec=grid_spec,
        out_shape=jax.ShapeDtypeStruct((B, DIM, NPOS), jnp.float32),
    )(starts, ns, nft, pm)
    return out.reshape(B, DIM, NROW, NROW)


# speedup vs baseline: 1.2700x; 1.1378x over previous
import jax
import jax.numpy as jnp
from jax import lax
from jax.experimental import pallas as pl
from jax.experimental.pallas import tpu as pltpu

DIM = 192
NPOS = 196
NROW = 14
NB = 16
GWIN = 3456
WINL = 384
_NCHUNK = GWIN // 128


def _body(s_ref, n_ref, nft_ref, pm_ref, out_ref, buf, sem):
    i = pl.program_id(0)
    ngroups = pl.num_programs(0)

    def issue(g, slot, wait):
        first = g * NB
        gbase = (s_ref[first] // 128) * 128
        need = (
            s_ref[first]
            - gbase
            + s_ref[first + NB - 1]
            - s_ref[first]
            + n_ref[first + NB - 1]
        )
        for c in range(_NCHUNK):
            cp = pltpu.make_async_copy(
                nft_ref.at[:, pl.ds(gbase + c * 128, 128)],
                buf.at[slot, :, pl.ds(c * 128, 128)],
                sem.at[slot],
            )

            @pl.when(jnp.logical_or(g < 2, jnp.logical_or(c == 0, c * 128 < need)))
            def _():
                if wait:
                    cp.wait()
                else:
                    cp.start()

    @pl.when(i == 0)
    def _():
        issue(0, 0, wait=False)

    @pl.when(i + 1 < ngroups)
    def _():
        issue(i + 1, (i + 1) % 2, wait=False)

    issue(i, i % 2, wait=True)

    slot = i % 2
    s_lane = lax.broadcasted_iota(jnp.int32, (1, NPOS), 1)
    pm = pm_ref[...]
    gbase128 = s_ref[i * NB] // 128
    for n in range(NB):
        b = i * NB + n
        start = s_ref[b]
        rel = (start // 128 - gbase128) * 128
        o = start - (start // 128) * 128
        w = buf[slot, :, pl.ds(rel, WINL)]
        g1 = pltpu.roll(w, lax.rem(WINL - o, WINL), axis=1)
        t = g1[:, :NPOS]
        t = t * (s_lane < n_ref[b]).astype(jnp.float32)
        o_b = lax.dot_general(
            t, pm, (((1,), (0,)), ((), ())), preferred_element_type=jnp.float32
        )
        out_ref[n] = o_b


def kernel(node_features, num_sp_list):
    ns = num_sp_list.astype(jnp.int32)
    starts = (jnp.cumsum(ns) - ns).astype(jnp.int32)
    B = ns.shape[0]
    nft = node_features.T
    jj = jnp.arange(NPOS, dtype=jnp.int32)
    pj = (jj % NROW) * NROW + jj // NROW
    pm = (jnp.arange(NPOS, dtype=jnp.int32)[:, None] == pj[None, :]).astype(
        jnp.float32
    )
    grid_spec = pltpu.PrefetchScalarGridSpec(
        num_scalar_prefetch=2,
        grid=(B // NB,),
        in_specs=[
            pl.BlockSpec(memory_space=pl.ANY),
            pl.BlockSpec((NPOS, NPOS), lambda i, s_ref, n_ref: (0, 0)),
        ],
        out_specs=pl.BlockSpec((NB, DIM, NPOS), lambda i, s_ref, n_ref: (i, 0, 0)),
        scratch_shapes=[
            pltpu.VMEM((2, DIM, GWIN), jnp.float32),
            pltpu.SemaphoreType.DMA((2,)),
        ],
    )
    out = pl.pallas_call(
        _body,
        grid_spec=grid_spec,
        out_shape=jax.ShapeDtypeStruct((B, DIM, NPOS), jnp.float32),
    )(starts, ns, nft, pm)
    return out.reshape(B, DIM, NROW, NROW)


# speedup vs baseline: 1.3490x; 1.0622x over previous
import jax
import jax.numpy as jnp
from jax import lax
from jax.experimental import pallas as pl
from jax.experimental.pallas import tpu as pltpu

DIM = 192
NPOS = 196
NROW = 14
NB = 32
GWIN = 6656
WINL = 384
_NCHUNK = GWIN // 128


def _body(s_ref, n_ref, nft_ref, pm_ref, out_ref, buf, sem):
    i = pl.program_id(0)
    ngroups = pl.num_programs(0)

    def issue(g, slot, wait):
        first = g * NB
        gbase = (s_ref[first] // 128) * 128
        need = (
            s_ref[first]
            - gbase
            + s_ref[first + NB - 1]
            - s_ref[first]
            + n_ref[first + NB - 1]
        )
        for c in range(_NCHUNK):
            cp = pltpu.make_async_copy(
                nft_ref.at[:, pl.ds(gbase + c * 128, 128)],
                buf.at[slot, :, pl.ds(c * 128, 128)],
                sem.at[slot],
            )

            @pl.when(jnp.logical_or(g < 2, jnp.logical_or(c == 0, c * 128 < need)))
            def _():
                if wait:
                    cp.wait()
                else:
                    cp.start()

    @pl.when(i == 0)
    def _():
        issue(0, 0, wait=False)

    @pl.when(i + 1 < ngroups)
    def _():
        issue(i + 1, (i + 1) % 2, wait=False)

    issue(i, i % 2, wait=True)

    slot = i % 2
    s_lane = lax.broadcasted_iota(jnp.int32, (1, NPOS), 1)
    pm = pm_ref[...]
    gbase128 = s_ref[i * NB] // 128
    for n in range(NB):
        b = i * NB + n
        start = s_ref[b]
        rel = (start // 128 - gbase128) * 128
        o = start - (start // 128) * 128
        w = buf[slot, :, pl.ds(rel, WINL)]
        g1 = pltpu.roll(w, lax.rem(WINL - o, WINL), axis=1)
        t = g1[:, :NPOS]
        t = t * (s_lane < n_ref[b]).astype(jnp.float32)
        o_b = lax.dot_general(
            t, pm, (((1,), (0,)), ((), ())), preferred_element_type=jnp.float32
        )
        out_ref[n] = o_b


def kernel(node_features, num_sp_list):
    ns = num_sp_list.astype(jnp.int32)
    starts = (jnp.cumsum(ns) - ns).astype(jnp.int32)
    B = ns.shape[0]
    nft = node_features.T
    jj = jnp.arange(NPOS, dtype=jnp.int32)
    pj = (jj % NROW) * NROW + jj // NROW
    pm = (jnp.arange(NPOS, dtype=jnp.int32)[:, None] == pj[None, :]).astype(
        jnp.float32
    )
    grid_spec = pltpu.PrefetchScalarGridSpec(
        num_scalar_prefetch=2,
        grid=(B // NB,),
        in_specs=[
            pl.BlockSpec(memory_space=pl.ANY),
            pl.BlockSpec((NPOS, NPOS), lambda i, s_ref, n_ref: (0, 0)),
        ],
        out_specs=pl.BlockSpec((NB, DIM, NPOS), lambda i, s_ref, n_ref: (i, 0, 0)),
        scratch_shapes=[
            pltpu.VMEM((2, DIM, GWIN), jnp.float32),
            pltpu.SemaphoreType.DMA((2,)),
        ],
    )
    out = pl.pallas_call(
        _body,
        grid_spec=grid_spec,
        out_shape=jax.ShapeDtypeStruct((B, DIM, NPOS), jnp.float32),
    )(starts, ns, nft, pm)
    return out.reshape(B, DIM, NROW, NROW)


# speedup vs baseline: 1.3606x; 1.0086x over previous
import jax
import jax.numpy as jnp
from jax import lax
from jax.experimental import pallas as pl
from jax.experimental.pallas import tpu as pltpu

DIM = 192
NPOS = 196
NROW = 14
NB = 64
GWIN = 12928
WINL = 384
_NCHUNK = GWIN // 128


def _body(s_ref, n_ref, nft_ref, pm_ref, out_ref, buf, sem):
    i = pl.program_id(0)
    ngroups = pl.num_programs(0)

    def issue(g, slot, wait):
        first = g * NB
        gbase = (s_ref[first] // 128) * 128
        need = (
            s_ref[first]
            - gbase
            + s_ref[first + NB - 1]
            - s_ref[first]
            + n_ref[first + NB - 1]
        )
        for c in range(_NCHUNK):
            cp = pltpu.make_async_copy(
                nft_ref.at[:, pl.ds(gbase + c * 128, 128)],
                buf.at[slot, :, pl.ds(c * 128, 128)],
                sem.at[slot],
            )

            @pl.when(jnp.logical_or(g < 2, jnp.logical_or(c == 0, c * 128 < need)))
            def _():
                if wait:
                    cp.wait()
                else:
                    cp.start()

    @pl.when(i == 0)
    def _():
        issue(0, 0, wait=False)

    @pl.when(i + 1 < ngroups)
    def _():
        issue(i + 1, (i + 1) % 2, wait=False)

    issue(i, i % 2, wait=True)

    slot = i % 2
    s_lane = lax.broadcasted_iota(jnp.int32, (1, NPOS), 1)
    pm = pm_ref[...]
    gbase128 = s_ref[i * NB] // 128
    for n in range(NB):
        b = i * NB + n
        start = s_ref[b]
        rel = (start // 128 - gbase128) * 128
        o = start - (start // 128) * 128
        w = buf[slot, :, pl.ds(rel, WINL)]
        g1 = pltpu.roll(w, lax.rem(WINL - o, WINL), axis=1)
        t = g1[:, :NPOS]
        t = t * (s_lane < n_ref[b]).astype(jnp.float32)
        o_b = lax.dot_general(
            t, pm, (((1,), (0,)), ((), ())), preferred_element_type=jnp.float32
        )
        out_ref[n] = o_b


def kernel(node_features, num_sp_list):
    ns = num_sp_list.astype(jnp.int32)
    starts = (jnp.cumsum(ns) - ns).astype(jnp.int32)
    B = ns.shape[0]
    nft = node_features.T
    jj = jnp.arange(NPOS, dtype=jnp.int32)
    pj = (jj % NROW) * NROW + jj // NROW
    pm = (jnp.arange(NPOS, dtype=jnp.int32)[:, None] == pj[None, :]).astype(
        jnp.float32
    )
    grid_spec = pltpu.PrefetchScalarGridSpec(
        num_scalar_prefetch=2,
        grid=(B // NB,),
        in_specs=[
            pl.BlockSpec(memory_space=pl.ANY),
            pl.BlockSpec((NPOS, NPOS), lambda i, s_ref, n_ref: (0, 0)),
        ],
        out_specs=pl.BlockSpec((NB, DIM, NPOS), lambda i, s_ref, n_ref: (i, 0, 0)),
        scratch_shapes=[
            pltpu.VMEM((2, DIM, GWIN), jnp.float32),
            pltpu.SemaphoreType.DMA((2,)),
        ],
    )
    out = pl.pallas_call(
        _body,
        grid_spec=grid_spec,
        out_shape=jax.ShapeDtypeStruct((B, DIM, NPOS), jnp.float32),
    )(starts, ns, nft, pm)
    return out.reshape(B, DIM, NROW, NROW)
